# baseline (speedup 1.0000x reference)
import functools

import jax
import jax.numpy as jnp
from jax.experimental import pallas as pl
from jax.experimental.pallas import tpu as pltpu

_TEMPERATURE = 0.07
_TOP_K = 2

_BM = 2048
_BN = 4096
_BK = 128


def _router_body(n_blocks, k_blocks,
                 x_ref, w_ref, be_ref, wr_ref, br_ref,
                 coeff_ref, tv_ref, ti_ref,
                 acc_ref, norms_ref):
    n = pl.program_id(1)
    k = pl.program_id(2)

    @pl.when(k == 0)
    def _():
        acc_ref[...] = jnp.dot(x_ref[...], w_ref[...],
                               preferred_element_type=jnp.float32)

    @pl.when(k > 0)
    def _():
        acc_ref[...] += jnp.dot(x_ref[...], w_ref[...],
                                preferred_element_type=jnp.float32)

    @pl.when(k == k_blocks - 1)
    def _():
        feat = jax.nn.gelu(acc_ref[...] + be_ref[...])
        part = jnp.dot(feat, wr_ref[...], preferred_element_type=jnp.float32)

        @pl.when(n == 0)
        def _():
            norms_ref[...] = part

        @pl.when(n > 0)
        def _():
            norms_ref[...] += part

        @pl.when(n == n_blocks - 1)
        def _():
            norms = norms_ref[...] + br_ref[...]
            nd = norms.shape[1]
            mean = jnp.mean(norms, axis=1, keepdims=True)
            var = jnp.sum((norms - mean) ** 2, axis=1, keepdims=True) / (nd - 1)
            std = jnp.sqrt(var) + 1e-6
            z = (norms - mean) / (std * _TEMPERATURE)
            z = z - jnp.max(z, axis=1, keepdims=True)
            e = jnp.exp(z)
            coeff = e / jnp.sum(e, axis=1, keepdims=True)
            coeff_ref[...] = coeff

            lane = jax.lax.broadcasted_iota(jnp.int32, coeff.shape, 1)
            v0 = jnp.max(coeff, axis=1, keepdims=True)
            i0 = jnp.min(jnp.where(coeff == v0, lane, nd), axis=1, keepdims=True)
            masked = jnp.where(lane == i0, -1.0, coeff)
            v1 = jnp.max(masked, axis=1, keepdims=True)
            i1 = jnp.min(jnp.where(masked == v1, lane, nd), axis=1, keepdims=True)
            tv_ref[...] = jnp.concatenate([v0, v1], axis=1)
            ti_ref[...] = jnp.concatenate([i0, i1], axis=1)


@jax.jit
def kernel(images, W_enc, b_enc, W_router, b_router):
    B, D = images.shape
    ND = W_router.shape[1]
    bm, bn, bk = min(_BM, B), min(_BN, D), min(_BK, D)
    m_blocks, n_blocks, k_blocks = B // bm, D // bn, D // bk

    body = functools.partial(_router_body, n_blocks, k_blocks)
    coeff, tv, ti = pl.pallas_call(
        body,
        grid=(m_blocks, n_blocks, k_blocks),
        in_specs=[
            pl.BlockSpec((bm, bk), lambda m, n, k: (m, k)),
            pl.BlockSpec((bk, bn), lambda m, n, k: (k, n)),
            pl.BlockSpec((1, bn), lambda m, n, k: (0, n)),
            pl.BlockSpec((bn, ND), lambda m, n, k: (n, 0)),
            pl.BlockSpec((1, ND), lambda m, n, k: (0, 0)),
        ],
        out_specs=[
            pl.BlockSpec((bm, ND), lambda m, n, k: (m, 0)),
            pl.BlockSpec((bm, _TOP_K), lambda m, n, k: (m, 0)),
            pl.BlockSpec((bm, _TOP_K), lambda m, n, k: (m, 0)),
        ],
        out_shape=[
            jax.ShapeDtypeStruct((B, ND), jnp.float32),
            jax.ShapeDtypeStruct((B, _TOP_K), jnp.float32),
            jax.ShapeDtypeStruct((B, _TOP_K), jnp.int32),
        ],
        scratch_shapes=[
            pltpu.VMEM((bm, bn), jnp.float32),
            pltpu.VMEM((bm, ND), jnp.float32),
        ],
        compiler_params=pltpu.CompilerParams(
            dimension_semantics=("parallel", "arbitrary", "arbitrary"),
        ),
    )(images, W_enc, b_enc.reshape(1, D), W_router, b_router.reshape(1, ND))
    return (coeff, tv, ti)


# speedup vs baseline: 1.7399x; 1.7399x over previous
import functools

import jax
import jax.numpy as jnp
from jax.experimental import pallas as pl
from jax.experimental.pallas import tpu as pltpu

_TEMPERATURE = 0.07
_TOP_K = 2

_BM = 2048
_BN = 2048
_BK = 512


def _router_body(n_blocks, k_blocks,
                 x_ref, w_ref, be_ref, wr_ref, br_ref,
                 coeff_ref, tv_ref, ti_ref,
                 acc_ref, norms_ref):
    n = pl.program_id(1)
    k = pl.program_id(2)

    @pl.when(k == 0)
    def _():
        acc_ref[...] = jnp.dot(x_ref[...], w_ref[...],
                               preferred_element_type=jnp.float32)

    @pl.when(k > 0)
    def _():
        acc_ref[...] += jnp.dot(x_ref[...], w_ref[...],
                                preferred_element_type=jnp.float32)

    @pl.when(k == k_blocks - 1)
    def _():
        feat = jax.nn.gelu(acc_ref[...] + be_ref[...])
        part = jnp.dot(feat, wr_ref[...], preferred_element_type=jnp.float32)

        @pl.when(n == 0)
        def _():
            norms_ref[...] = part

        @pl.when(n > 0)
        def _():
            norms_ref[...] += part

        @pl.when(n == n_blocks - 1)
        def _():
            norms = norms_ref[...] + br_ref[...]
            nd = norms.shape[1]
            mean = jnp.mean(norms, axis=1, keepdims=True)
            var = jnp.sum((norms - mean) ** 2, axis=1, keepdims=True) / (nd - 1)
            std = jnp.sqrt(var) + 1e-6
            z = (norms - mean) / (std * _TEMPERATURE)
            z = z - jnp.max(z, axis=1, keepdims=True)
            e = jnp.exp(z)
            coeff = e / jnp.sum(e, axis=1, keepdims=True)
            coeff_ref[...] = coeff

            lane = jax.lax.broadcasted_iota(jnp.int32, coeff.shape, 1)
            v0 = jnp.max(coeff, axis=1, keepdims=True)
            i0 = jnp.min(jnp.where(coeff == v0, lane, nd), axis=1, keepdims=True)
            masked = jnp.where(lane == i0, -1.0, coeff)
            v1 = jnp.max(masked, axis=1, keepdims=True)
            i1 = jnp.min(jnp.where(masked == v1, lane, nd), axis=1, keepdims=True)
            tv_ref[...] = jnp.concatenate([v0, v1], axis=1)
            ti_ref[...] = jnp.concatenate([i0, i1], axis=1)


@jax.jit
def kernel(images, W_enc, b_enc, W_router, b_router):
    B, D = images.shape
    ND = W_router.shape[1]
    bm, bn, bk = min(_BM, B), min(_BN, D), min(_BK, D)
    m_blocks, n_blocks, k_blocks = B // bm, D // bn, D // bk

    body = functools.partial(_router_body, n_blocks, k_blocks)
    coeff, tv, ti = pl.pallas_call(
        body,
        grid=(m_blocks, n_blocks, k_blocks),
        in_specs=[
            pl.BlockSpec((bm, bk), lambda m, n, k: (m, k)),
            pl.BlockSpec((bk, bn), lambda m, n, k: (k, n)),
            pl.BlockSpec((1, bn), lambda m, n, k: (0, n)),
            pl.BlockSpec((bn, ND), lambda m, n, k: (n, 0)),
            pl.BlockSpec((1, ND), lambda m, n, k: (0, 0)),
        ],
        out_specs=[
            pl.BlockSpec((bm, ND), lambda m, n, k: (m, 0)),
            pl.BlockSpec((bm, _TOP_K), lambda m, n, k: (m, 0)),
            pl.BlockSpec((bm, _TOP_K), lambda m, n, k: (m, 0)),
        ],
        out_shape=[
            jax.ShapeDtypeStruct((B, ND), jnp.float32),
            jax.ShapeDtypeStruct((B, _TOP_K), jnp.float32),
            jax.ShapeDtypeStruct((B, _TOP_K), jnp.int32),
        ],
        scratch_shapes=[
            pltpu.VMEM((bm, bn), jnp.float32),
            pltpu.VMEM((bm, ND), jnp.float32),
        ],
        compiler_params=pltpu.CompilerParams(
            dimension_semantics=("parallel", "arbitrary", "arbitrary"),
        ),
    )(images, W_enc, b_enc.reshape(1, D), W_router, b_router.reshape(1, ND))
    return (coeff, tv, ti)
